# Initial kernel scaffold; baseline (speedup 1.0000x reference)
import jax
import jax.numpy as jnp
from jax.experimental import pallas as pl


def kernel(x, edge_index_s0, edge_index_s1, edge_index_s2, params):
    raise NotImplementedError("write your pallas kernel here")


# speedup vs baseline: 1.6893x; 1.6893x over previous
import functools
import jax
import jax.numpy as jnp
from jax.experimental import pallas as pl

N = 50000
RS = 0.9999950000374996


def _head_kernel(fused_ref, wf_ref, bf_ref, wc_ref, bc_ref, wr_ref, br_ref,
                 logits_ref, reg_ref):
    fused = fused_ref[...]
    h = jnp.maximum(fused @ wf_ref[...] + bf_ref[...], 0.0)
    logits_ref[...] = h @ wc_ref[...] + bc_ref[...]
    reg_ref[...] = jax.nn.sigmoid(h @ wr_ref[...] + br_ref[...])


def _head(fused, f):
    return pl.pallas_call(
        _head_kernel,
        out_shape=(jax.ShapeDtypeStruct((1, 10), jnp.float32),
                   jax.ShapeDtypeStruct((1, 1), jnp.float32)),
    )(fused, f["Wf"], f["bf"][None, :], f["Wc"], f["bc"][None, :],
      f["Wr"], f["br"][None, :])


def _branch(x, ei, layers):
    src, dst = ei[0], ei[1]
    ones = jnp.ones(src.shape[0], jnp.float32)
    deg = jax.ops.segment_sum(ones, dst, num_segments=N) + 1.0
    dinv = jax.lax.rsqrt(deg)
    w = jax.ops.segment_sum(dinv[dst], src, num_segments=N)
    v = (w + dinv) * dinv * (1.0 / N)

    h = x
    for i in range(2):
        p = layers[i]
        g = dinv[:, None] * (h @ p["W"])
        s = jax.ops.segment_sum(g[src], dst, num_segments=N) + g
        c1 = RS * p["gamma"]
        h = jnp.maximum((dinv[:, None] * s + p["b"]) * c1 + p["beta"], 0.0)

    p = layers[2]
    u = v @ h
    c1 = RS * p["gamma"]
    return (u @ p["W"]) * c1 + c1 * p["b"] + p["beta"]


def kernel(x, edge_index_s0, edge_index_s1, edge_index_s2, params):
    eis = [edge_index_s0, edge_index_s1, edge_index_s2]
    embs = [_branch(x, eis[i], params["scales"][i]) for i in range(3)]
    fused = jnp.concatenate(embs)[None, :]
    logits, reg = _head(fused, params["fusion"])
    return (logits, reg)


# speedup vs baseline: 2.3770x; 1.4071x over previous
import functools

import jax
import jax.numpy as jnp
from jax import lax
from jax.experimental import pallas as pl
from jax.experimental.pallas import tpu as pltpu
from jax.experimental.pallas import tpu_sc as plsc

N = 50000
RS = 0.9999950000374996

L = 128
K = 8
CH = K * L
W = 16
NP = 50048
RPT = NP // 16
DUMP = N


@functools.lru_cache(maxsize=None)
def _agg_call(P, CHUNKS):
    mesh = plsc.VectorSubcoreMesh(core_axis_name="c", subcore_axis_name="s")
    out_type = tuple(jax.ShapeDtypeStruct((NP, W), jnp.float32)
                     for _ in range(P))
    scratch = [
        pltpu.VMEM((K, L), jnp.int32),
        pltpu.VMEM((K, L), jnp.int32),
        pltpu.VMEM((CH, W), jnp.float32),
        pltpu.VMEM_SHARED((NP, W), jnp.float32),
        pltpu.SemaphoreType.DMA,
    ]

    def body(src_hbm, dst_hbm, *rest):
        g_refs = rest[:P]
        s_refs = rest[P:2 * P]
        src_v, dst_v, rows_v, acc, sem = rest[2 * P:]
        cid = lax.axis_index("c")
        sid = lax.axis_index("s")
        for p in range(P):
            @pl.when(cid == (p % 2))
            def _(p=p):
                g = g_refs[p]
                pltpu.sync_copy(g.at[pl.ds(sid * RPT, RPT)],
                                acc.at[pl.ds(sid * RPT, RPT)])
                plsc.subcore_barrier()

                def chunk(j, carry):
                    r = sid * CHUNKS + j
                    pltpu.sync_copy(src_hbm.at[r], src_v)
                    pltpu.sync_copy(dst_hbm.at[r], dst_v)
                    handles = [
                        pltpu.async_copy(g.at[src_v.at[k]],
                                         rows_v.at[pl.ds(k * L, L)], sem)
                        for k in range(K)
                    ]
                    for h in handles:
                        h.wait()
                    for k in range(K):
                        pltpu.sync_copy(rows_v.at[pl.ds(k * L, L)],
                                        acc.at[dst_v.at[k]], add=True)
                    return carry

                lax.fori_loop(0, CHUNKS, chunk, 0)
                plsc.subcore_barrier()
                pltpu.sync_copy(acc.at[pl.ds(sid * RPT, RPT)],
                                s_refs[p].at[pl.ds(sid * RPT, RPT)])
                plsc.subcore_barrier()

    return pl.kernel(body, out_type=out_type, mesh=mesh,
                     scratch_types=scratch,
                     compiler_params=pltpu.CompilerParams(
                         use_tc_tiling_on_sc=False))


def _pad_edges(ei):
    e = ei.shape[1]
    per = -(-e // (16 * CH)) * CH
    chunks = per // CH
    pad = 16 * per - e
    src = jnp.concatenate([ei[0], jnp.zeros((pad,), jnp.int32)])
    dst = jnp.concatenate([ei[1], jnp.full((pad,), DUMP, jnp.int32)])
    return (src.reshape(16 * chunks, K, L), dst.reshape(16 * chunks, K, L),
            chunks)


def _head_kernel(fused_ref, wf_ref, bf_ref, wc_ref, bc_ref, wr_ref, br_ref,
                 logits_ref, reg_ref):
    fused = fused_ref[...]
    h = jnp.maximum(fused @ wf_ref[...] + bf_ref[...], 0.0)
    logits_ref[...] = h @ wc_ref[...] + bc_ref[...]
    reg_ref[...] = jax.nn.sigmoid(h @ wr_ref[...] + br_ref[...])


def _head(fused, f):
    return pl.pallas_call(
        _head_kernel,
        out_shape=(jax.ShapeDtypeStruct((1, 10), jnp.float32),
                   jax.ShapeDtypeStruct((1, 1), jnp.float32)),
    )(fused, f["Wf"], f["bf"][None, :], f["Wc"], f["bc"][None, :],
      f["Wr"], f["br"][None, :])


def _branch(x, ei, layers):
    src, dst = ei[0], ei[1]
    ones = jnp.ones(src.shape[0], jnp.float32)
    deg = jax.ops.segment_sum(ones, dst, num_segments=N) + 1.0
    dinv = jax.lax.rsqrt(deg)
    w = jax.ops.segment_sum(dinv[dst], src, num_segments=N)
    v = (w + dinv) * dinv * (1.0 / N)

    src3, dst3, chunks = _pad_edges(ei)

    h = x
    for i in range(2):
        p = layers[i]
        g = dinv[:, None] * (h @ p["W"])
        f_dim = g.shape[1]
        slices = f_dim // W
        gp = jnp.pad(g, ((0, NP - N), (0, 0)))
        g_sl = [gp[:, W * q:W * q + W] for q in range(slices)]
        s_sl = _agg_call(slices, chunks)(src3, dst3, *g_sl)
        s = jnp.concatenate([t[:N] for t in s_sl], axis=1)
        c1 = RS * p["gamma"]
        h = jnp.maximum((dinv[:, None] * s + p["b"]) * c1 + p["beta"], 0.0)

    p = layers[2]
    u = v @ h
    c1 = RS * p["gamma"]
    return (u @ p["W"]) * c1 + c1 * p["b"] + p["beta"]


def kernel(x, edge_index_s0, edge_index_s1, edge_index_s2, params):
    eis = [edge_index_s0, edge_index_s1, edge_index_s2]
    embs = [_branch(x, eis[i], params["scales"][i]) for i in range(3)]
    fused = jnp.concatenate(embs)[None, :]
    logits, reg = _head(fused, params["fusion"])
    return (logits, reg)
